# baseline (speedup 1.0000x reference)
import jax
import jax.numpy as jnp
from jax import lax
from jax.experimental import pallas as pl
from jax.experimental.pallas import tpu as pltpu
from jax.experimental.pallas import tpu_sc as plsc

N_FIELDS = 26
VOCAB = 100000
EMB_DIM = 50
EMB_PAD = 64
BATCH = 16384
TOTAL_ROWS = BATCH * N_FIELDS

NUM_WORKERS = 32
ROWS_PER_WORKER = TOTAL_ROWS // NUM_WORKERS
CHUNK = 512
SUBGATHER = 128
NSUB = CHUNK // SUBGATHER
NCH = ROWS_PER_WORKER // CHUNK
NBUF = 2


def _sc_gather_body(tables_hbm, idx_hbm, out_hbm, idx_all, bufs, sems):
    wid = lax.axis_index("s") * 2 + lax.axis_index("c")
    base = wid * ROWS_PER_WORKER
    gsems = sems[:NBUF]
    wsems = sems[NBUF:]

    pltpu.sync_copy(idx_hbm.at[pl.ds(base, ROWS_PER_WORKER)], idx_all)

    def issue_gather(c, b):
        for k in range(NSUB):
            pltpu.async_copy(
                tables_hbm.at[idx_all.at[pl.ds(c * CHUNK + k * SUBGATHER,
                                               SUBGATHER)]],
                bufs[b].at[pl.ds(k * SUBGATHER, SUBGATHER)],
                gsems[b])

    def wait_gather(b):
        pltpu.make_async_copy(tables_hbm.at[pl.ds(0, CHUNK)], bufs[b],
                              gsems[b]).wait()

    def write_out(c, b):
        dst = out_hbm.at[pl.ds(base + c * CHUNK, CHUNK)]
        pltpu.async_copy(bufs[b], dst, wsems[b])
        pltpu.make_async_copy(bufs[b], dst, wsems[b]).wait()

    for b in range(NBUF):
        issue_gather(b, b)

    def step(g0, carry):
        for b in range(NBUF):
            c = g0 * NBUF + b
            wait_gather(b)
            write_out(c, b)
            issue_gather(c + NBUF, b)
        return carry

    lax.fori_loop(0, (NCH - NBUF) // NBUF, step, 0)

    for b in range(NBUF):
        c = NCH - NBUF + b
        wait_gather(b)
        write_out(c, b)


_sc_gather = pl.kernel(
    _sc_gather_body,
    out_type=jax.ShapeDtypeStruct((TOTAL_ROWS, EMB_PAD), jnp.float32),
    mesh=plsc.VectorSubcoreMesh(core_axis_name="c", subcore_axis_name="s"),
    scratch_types=[
        pltpu.VMEM((ROWS_PER_WORKER,), jnp.int32),
        [pltpu.VMEM((CHUNK, EMB_PAD), jnp.float32) for _ in range(NBUF)],
        [pltpu.SemaphoreType.DMA for _ in range(2 * NBUF)],
    ],
    compiler_params=pltpu.CompilerParams(use_tc_tiling_on_sc=False),
)


_VCHUNK = 25088


def _xpose_body(t_ref, eye_ref, out_ref):
    out_ref[0] = jax.lax.dot_general(
        t_ref[0], eye_ref[...], (((0,), (0,)), ((), ())),
        preferred_element_type=jnp.float32)


def _pad_tables(tables):
    t_t = jnp.transpose(tables, (0, 2, 1))
    eye = jnp.eye(EMB_DIM, EMB_PAD, dtype=jnp.float32)
    grid = (N_FIELDS, (VOCAB + _VCHUNK - 1) // _VCHUNK)
    out = pl.pallas_call(
        _xpose_body,
        grid=grid,
        in_specs=[
            pl.BlockSpec((1, EMB_DIM, _VCHUNK), lambda f, j: (f, 0, j)),
            pl.BlockSpec((EMB_DIM, EMB_PAD), lambda f, j: (0, 0)),
        ],
        out_specs=pl.BlockSpec((1, _VCHUNK, EMB_PAD), lambda f, j: (f, j, 0)),
        out_shape=jax.ShapeDtypeStruct((N_FIELDS, VOCAB, EMB_PAD),
                                       jnp.float32),
    )(t_t, eye)
    return out.reshape(N_FIELDS * VOCAB, EMB_PAD)


def _mlp_body(g_ref, xn_ref, w1a_ref, w1b_ref, b1_ref, w2_ref, b2_ref,
              w3_ref, b3_ref, out_ref):
    h = jnp.dot(g_ref[...], w1a_ref[...], preferred_element_type=jnp.float32)
    h += jnp.dot(xn_ref[...], w1b_ref[...], preferred_element_type=jnp.float32)
    h = jnp.maximum(h + b1_ref[...], 0.0)
    h = jnp.maximum(
        jnp.dot(h, w2_ref[...], preferred_element_type=jnp.float32)
        + b2_ref[...], 0.0)
    out_ref[...] = (
        jnp.dot(h, w3_ref[...], preferred_element_type=jnp.float32)
        + b3_ref[...])


def _mlp(g, xn, w1a, w1b, b1, w2, b2, w3, b3, block_rows=1024):
    grid = (BATCH // block_rows,)
    return pl.pallas_call(
        _mlp_body,
        grid=grid,
        in_specs=[
            pl.BlockSpec((block_rows, g.shape[1]), lambda i: (i, 0)),
            pl.BlockSpec((block_rows, xn.shape[1]), lambda i: (i, 0)),
            pl.BlockSpec(w1a.shape, lambda i: (0, 0)),
            pl.BlockSpec(w1b.shape, lambda i: (0, 0)),
            pl.BlockSpec(b1.shape, lambda i: (0, 0)),
            pl.BlockSpec(w2.shape, lambda i: (0, 0)),
            pl.BlockSpec(b2.shape, lambda i: (0, 0)),
            pl.BlockSpec(w3.shape, lambda i: (0, 0)),
            pl.BlockSpec(b3.shape, lambda i: (0, 0)),
        ],
        out_specs=pl.BlockSpec((block_rows, 1), lambda i: (i, 0)),
        out_shape=jax.ShapeDtypeStruct((BATCH, 1), jnp.float32),
    )(g, xn, w1a, w1b, b1, w2, b2, w3, b3)


@jax.jit
def kernel(x_cat, x_num, tables, W1, b1, W2, b2, W3, b3):
    tables56 = _pad_tables(tables)
    idx = (x_cat + jnp.arange(N_FIELDS, dtype=jnp.int32) * VOCAB).reshape(-1)
    rows = _sc_gather(tables56, idx)
    g = rows.reshape(BATCH, N_FIELDS * EMB_PAD)
    w1a = jnp.pad(W1[: N_FIELDS * EMB_DIM].reshape(N_FIELDS, EMB_DIM, -1),
                  ((0, 0), (0, EMB_PAD - EMB_DIM), (0, 0)))
    w1a = w1a.reshape(N_FIELDS * EMB_PAD, -1)
    w1b = W1[N_FIELDS * EMB_DIM:]
    return _mlp(g, x_num, w1a, w1b, b1.reshape(1, -1), W2,
                b2.reshape(1, -1), W3, b3.reshape(1, -1))


# speedup vs baseline: 1.7318x; 1.7318x over previous
import jax
import jax.numpy as jnp
from jax import lax
from jax.experimental import pallas as pl
from jax.experimental.pallas import tpu as pltpu
from jax.experimental.pallas import tpu_sc as plsc

N_FIELDS = 26
VOCAB = 100000
EMB_DIM = 50
EMB_PAD = 128
BATCH = 16384
TOTAL_ROWS = BATCH * N_FIELDS

NUM_WORKERS = 32
ROWS_PER_WORKER = TOTAL_ROWS // NUM_WORKERS
CHUNK = 256
SUBGATHER = 128
NSUB = CHUNK // SUBGATHER
NCH = ROWS_PER_WORKER // CHUNK
NBUF = 2


def _sc_gather_body(tables_hbm, idx_hbm, out_hbm, idx_all, bufs, sems):
    wid = lax.axis_index("s") * 2 + lax.axis_index("c")
    base = wid * ROWS_PER_WORKER
    gsems = sems[:NBUF]
    wsems = sems[NBUF:]

    pltpu.sync_copy(idx_hbm.at[pl.ds(base, ROWS_PER_WORKER)], idx_all)

    def issue_gather(c, b):
        for k in range(NSUB):
            pltpu.async_copy(
                tables_hbm.at[idx_all.at[pl.ds(c * CHUNK + k * SUBGATHER,
                                               SUBGATHER)]],
                bufs[b].at[pl.ds(k * SUBGATHER, SUBGATHER)],
                gsems[b])

    def wait_gather(b):
        pltpu.make_async_copy(tables_hbm.at[pl.ds(0, CHUNK)], bufs[b],
                              gsems[b]).wait()

    def write_out(c, b):
        dst = out_hbm.at[pl.ds(base + c * CHUNK, CHUNK)]
        pltpu.async_copy(bufs[b], dst, wsems[b])
        pltpu.make_async_copy(bufs[b], dst, wsems[b]).wait()

    for b in range(NBUF):
        issue_gather(b, b)

    def step(g0, carry):
        for b in range(NBUF):
            c = g0 * NBUF + b
            wait_gather(b)
            write_out(c, b)
            issue_gather(c + NBUF, b)
        return carry

    lax.fori_loop(0, (NCH - NBUF) // NBUF, step, 0)

    for b in range(NBUF):
        c = NCH - NBUF + b
        wait_gather(b)
        write_out(c, b)


_sc_gather = pl.kernel(
    _sc_gather_body,
    out_type=jax.ShapeDtypeStruct((TOTAL_ROWS, EMB_PAD), jnp.float32),
    mesh=plsc.VectorSubcoreMesh(core_axis_name="c", subcore_axis_name="s"),
    scratch_types=[
        pltpu.VMEM((ROWS_PER_WORKER,), jnp.int32),
        [pltpu.VMEM((CHUNK, EMB_PAD), jnp.float32) for _ in range(NBUF)],
        [pltpu.SemaphoreType.DMA for _ in range(2 * NBUF)],
    ],
    compiler_params=pltpu.CompilerParams(use_tc_tiling_on_sc=False),
)


_VCHUNK = 25088


def _xpose_body(t_ref, eye_ref, out_ref):
    out_ref[0] = jax.lax.dot_general(
        t_ref[0], eye_ref[...], (((0,), (0,)), ((), ())),
        preferred_element_type=jnp.float32)


def _pad_tables(tables):
    t_t = jnp.transpose(tables, (0, 2, 1))
    eye = jnp.eye(EMB_DIM, EMB_PAD, dtype=jnp.float32)
    grid = (N_FIELDS, (VOCAB + _VCHUNK - 1) // _VCHUNK)
    out = pl.pallas_call(
        _xpose_body,
        grid=grid,
        in_specs=[
            pl.BlockSpec((1, EMB_DIM, _VCHUNK), lambda f, j: (f, 0, j)),
            pl.BlockSpec((EMB_DIM, EMB_PAD), lambda f, j: (0, 0)),
        ],
        out_specs=pl.BlockSpec((1, _VCHUNK, EMB_PAD), lambda f, j: (f, j, 0)),
        out_shape=jax.ShapeDtypeStruct((N_FIELDS, VOCAB, EMB_PAD),
                                       jnp.float32),
    )(t_t, eye)
    return out.reshape(N_FIELDS * VOCAB, EMB_PAD)


def _mlp_body(g_ref, xn_ref, w1a_ref, w1b_ref, b1_ref, w2_ref, b2_ref,
              w3_ref, b3_ref, out_ref):
    h = jnp.dot(g_ref[...], w1a_ref[...], preferred_element_type=jnp.float32)
    h += jnp.dot(xn_ref[...], w1b_ref[...], preferred_element_type=jnp.float32)
    h = jnp.maximum(h + b1_ref[...], 0.0)
    h = jnp.maximum(
        jnp.dot(h, w2_ref[...], preferred_element_type=jnp.float32)
        + b2_ref[...], 0.0)
    out_ref[...] = (
        jnp.dot(h, w3_ref[...], preferred_element_type=jnp.float32)
        + b3_ref[...])


def _mlp(g, xn, w1a, w1b, b1, w2, b2, w3, b3, block_rows=1024):
    grid = (BATCH // block_rows,)
    return pl.pallas_call(
        _mlp_body,
        grid=grid,
        in_specs=[
            pl.BlockSpec((block_rows, g.shape[1]), lambda i: (i, 0)),
            pl.BlockSpec((block_rows, xn.shape[1]), lambda i: (i, 0)),
            pl.BlockSpec(w1a.shape, lambda i: (0, 0)),
            pl.BlockSpec(w1b.shape, lambda i: (0, 0)),
            pl.BlockSpec(b1.shape, lambda i: (0, 0)),
            pl.BlockSpec(w2.shape, lambda i: (0, 0)),
            pl.BlockSpec(b2.shape, lambda i: (0, 0)),
            pl.BlockSpec(w3.shape, lambda i: (0, 0)),
            pl.BlockSpec(b3.shape, lambda i: (0, 0)),
        ],
        out_specs=pl.BlockSpec((block_rows, 1), lambda i: (i, 0)),
        out_shape=jax.ShapeDtypeStruct((BATCH, 1), jnp.float32),
    )(g, xn, w1a, w1b, b1, w2, b2, w3, b3)


@jax.jit
def kernel(x_cat, x_num, tables, W1, b1, W2, b2, W3, b3):
    tables_pad = _pad_tables(tables)
    idx = (x_cat + jnp.arange(N_FIELDS, dtype=jnp.int32) * VOCAB).reshape(-1)
    rows = _sc_gather(tables_pad, idx)
    g = rows.reshape(BATCH, N_FIELDS * EMB_PAD)
    w1a = jnp.pad(W1[: N_FIELDS * EMB_DIM].reshape(N_FIELDS, EMB_DIM, -1),
                  ((0, 0), (0, EMB_PAD - EMB_DIM), (0, 0)))
    w1a = w1a.reshape(N_FIELDS * EMB_PAD, -1)
    w1b = W1[N_FIELDS * EMB_DIM:]
    return _mlp(g, x_num, w1a, w1b, b1.reshape(1, -1), W2,
                b2.reshape(1, -1), W3, b3.reshape(1, -1))
